# baseline (speedup 1.0000x reference)
import functools

import jax
import jax.numpy as jnp
from jax import lax
from jax.experimental import pallas as pl
from jax.experimental.pallas import tpu as pltpu
from jax.experimental.pallas import tpu_sc as plsc

_NC = 2
_NS = 16


def _make_segsum(N, NP, E, w, K, NBUF):
    NW = _NC * _NS
    EPW = E // NW
    C = EPW // K
    assert EPW * NW == E and C * K == EPW and K % 8 == 0 and C >= NBUF
    RPT = NP // _NS
    assert RPT * _NS == NP and RPT % 8 == 0
    mesh = plsc.VectorSubcoreMesh(core_axis_name="c", subcore_axis_name="s")

    @functools.partial(
        pl.kernel,
        out_type=jax.ShapeDtypeStruct((NP, 128), jnp.float32),
        mesh=mesh,
        compiler_params=pltpu.CompilerParams(use_tc_tiling_on_sc=False),
        scratch_types=[
            pltpu.VMEM((C, K), jnp.int32),
            pltpu.VMEM((C, K), jnp.int32),
            pltpu.VMEM((NBUF, K, w), jnp.float32),
            pltpu.VMEM_SHARED((NP, w), jnp.float32),
        ] + [pltpu.SemaphoreType.DMA] * (2 * NBUF),
    )
    def segsum(table, src_idx, dst_idx, zeros, out, src_v, dst_v, rows_v,
               acc_sh, *sems):
        gsem = sems[:NBUF]
        ssem = sems[NBUF:]
        c = lax.axis_index("c")
        s = lax.axis_index("s")
        wid = c * _NS + s
        r0 = s * RPT
        zc = pltpu.async_copy(zeros.at[pl.ds(0, RPT), pl.ds(0, w)],
                              acc_sh.at[pl.ds(r0, RPT)], ssem[0])
        sc0 = pltpu.async_copy(src_idx.at[wid], src_v, gsem[0])
        dc0 = pltpu.async_copy(dst_idx.at[wid], dst_v, gsem[1])
        sc0.wait()
        dc0.wait()

        def gstart(b, i):
            pltpu.async_copy(table.at[src_v.at[i]], rows_v.at[b], gsem[b])

        def swait(b, i):
            pltpu.make_async_copy(rows_v.at[b], acc_sh.at[dst_v.at[i]],
                                  ssem[b]).wait()

        def step(b, i):
            pltpu.make_async_copy(table.at[src_v.at[i]], rows_v.at[b],
                                  gsem[b]).wait()
            pltpu.async_copy(rows_v.at[b], acc_sh.at[dst_v.at[i]],
                             ssem[b], add=True)
            nb = (b + 2) % NBUF

            @pl.when(i + 2 < C)
            def _():
                @pl.when(i >= NBUF - 2)
                def _():
                    swait(nb, i)

                gstart(nb, i + 2)

        gstart(0, 0)
        gstart(1, 1)
        zc.wait()
        plsc.subcore_barrier()

        @pl.loop(0, C // NBUF)
        def _grp(j):
            i0 = NBUF * j
            for t in range(NBUF):
                step(t, i0 + t)

        for t in range(C % NBUF):
            step(t, C - (C % NBUF) + t)

        for b in range(NBUF):
            swait(b, 0)

        plsc.subcore_barrier()
        pltpu.sync_copy(acc_sh.at[pl.ds(r0, RPT)],
                        out.at[pl.ds(r0, RPT), pl.ds(c * w, w)])

    return segsum


_N, _NP, _E = 10000, 10240, 320000
_R = 10000


def _make_l1pre(N, D, dout2, R):
    def body(x, wcat, out):
        out[...] = jnp.dot(x[...], wcat[...],
                           preferred_element_type=jnp.float32)

    return pl.pallas_call(
        body,
        grid=(N // R,),
        in_specs=[
            pl.BlockSpec((R, D), lambda i: (i, 0)),
            pl.BlockSpec((D, dout2), lambda i: (0, 0)),
        ],
        out_specs=pl.BlockSpec((R, dout2), lambda i: (i, 0)),
        out_shape=jax.ShapeDtypeStruct((N, dout2), jnp.float32),
    )


def _make_l1post(N, NP, dout, R):
    def body(a, pr, b, out):
        out[...] = (a[:, :dout] + a[:, dout:2 * dout]
                    + pr[:, dout:2 * dout] + b[...])

    return pl.pallas_call(
        body,
        grid=(N // R,),
        in_specs=[
            pl.BlockSpec((R, 128), lambda i: (i, 0)),
            pl.BlockSpec((R, 2 * dout), lambda i: (i, 0)),
            pl.BlockSpec((1, dout), lambda i: (0, 0)),
        ],
        out_specs=pl.BlockSpec((R, dout), lambda i: (i, 0)),
        out_shape=jax.ShapeDtypeStruct((N, dout), jnp.float32),
    )


def _make_post(N, NP, din, dout, R):
    def body(a, h, wrel2, wroot, b, out):
        out[...] = (jnp.dot(a[:, :2 * din], wrel2[...],
                            preferred_element_type=jnp.float32)
                    + jnp.dot(h[...], wroot[...],
                              preferred_element_type=jnp.float32)
                    + b[...])

    return pl.pallas_call(
        body,
        grid=(N // R,),
        in_specs=[
            pl.BlockSpec((R, 128), lambda i: (i, 0)),
            pl.BlockSpec((R, din), lambda i: (i, 0)),
            pl.BlockSpec((2 * din, dout), lambda i: (0, 0)),
            pl.BlockSpec((din, dout), lambda i: (0, 0)),
            pl.BlockSpec((1, dout), lambda i: (0, 0)),
        ],
        out_specs=pl.BlockSpec((R, dout), lambda i: (i, 0)),
        out_shape=jax.ShapeDtypeStruct((N, dout), jnp.float32),
    )


def _make_post5_pool(N, din, G, C, R):
    nblk = N // R
    D = 2 * din

    def body(a, h, wrel2, wroot, b, batch, wlin, blin, out, acc):
        i = pl.program_id(0)

        @pl.when(i == 0)
        def _():
            acc[...] = jnp.zeros_like(acc)

        h5 = (jnp.dot(a[...], wrel2[...], preferred_element_type=jnp.float32)
              + jnp.dot(h[...], wroot[...],
                        preferred_element_type=jnp.float32)
              + b[...])
        onehot = (lax.broadcasted_iota(jnp.int32, (G, R), 0)
                  == batch[0]).astype(jnp.float32)
        acc[...] += jnp.dot(onehot, h5, preferred_element_type=jnp.float32)

        @pl.when(i == nblk - 1)
        def _():
            out[...] = (jnp.dot(acc[...], wlin[...],
                                preferred_element_type=jnp.float32)
                        + blin[...])

    return pl.pallas_call(
        body,
        grid=(nblk,),
        in_specs=[
            pl.BlockSpec((R, 128), lambda i: (i, 0)),
            pl.BlockSpec((R, din), lambda i: (i, 0)),
            pl.BlockSpec((D, D), lambda i: (0, 0)),
            pl.BlockSpec((din, D), lambda i: (0, 0)),
            pl.BlockSpec((1, D), lambda i: (0, 0)),
            pl.BlockSpec((1, 1, R), lambda i: (i, 0, 0)),
            pl.BlockSpec((D, C), lambda i: (0, 0)),
            pl.BlockSpec((1, C), lambda i: (0, 0)),
        ],
        out_specs=pl.BlockSpec((G, C), lambda i: (0, 0)),
        out_shape=jax.ShapeDtypeStruct((G, C), jnp.float32),
        scratch_shapes=[pltpu.VMEM((G, D), jnp.float32)],
    )
_K_w = {8: (2000, 2), 16: (2000, 2), 32: (1000, 2), 64: (200, 3)}
_segsum_w = {w: _make_segsum(_N, _NP, _E, w, k, nb)
             for w, (k, nb) in _K_w.items()}
_l1pre = _make_l1pre(_N, 128, 16, _R)
_l1post = _make_l1post(_N, _NP, 8, _R)
_post_w = {
    8: _make_post(_N, _NP, 8, 16, _R),
    16: _make_post(_N, _NP, 16, 32, _R),
    32: _make_post(_N, _NP, 32, 64, _R),
}
_post5_pool = _make_post5_pool(_N, 64, 256, 10, _R)


def kernel(x, edge_index, batch, W_rel1, b_rel1, W_root1, W_rel2, b_rel2,
           W_root2, W_rel3, b_rel3, W_root3, W_rel4, b_rel4, W_root4,
           W_rel5, b_rel5, W_root5, W_lin, b_lin):
    idx3 = {}
    for w, (k, _nb) in _K_w.items():
        c = _E // 32 // k
        idx3[w] = (edge_index[0].reshape(32, c, k),
                   edge_index[1].reshape(32, c, k))
    zeros = jnp.zeros((_NP // _NS, 128), jnp.float32)

    pr = _l1pre(x, jnp.concatenate([W_rel1, W_root1], axis=1))
    A = _segsum_w[8](pr[:, :8], idx3[8][0], idx3[8][1], zeros)
    h = _l1post(A, pr, b_rel1.reshape(1, -1))

    for (W_rel, b_rel, W_root) in (
        (W_rel2, b_rel2, W_root2),
        (W_rel3, b_rel3, W_root3),
        (W_rel4, b_rel4, W_root4),
    ):
        w = h.shape[1]
        A = _segsum_w[w](h, idx3[w][0], idx3[w][1], zeros)
        wrel2 = jnp.concatenate([W_rel, W_rel], axis=0)
        h = _post_w[w](A, h, wrel2, W_root, b_rel.reshape(1, -1))

    A = _segsum_w[64](h, idx3[64][0], idx3[64][1], zeros)
    wrel2 = jnp.concatenate([W_rel5, W_rel5], axis=0)
    return _post5_pool(A, h, wrel2, W_root5, b_rel5.reshape(1, -1),
                       batch.reshape(_N // _R, 1, _R), W_lin,
                       b_lin.reshape(1, -1))


# speedup vs baseline: 1.0201x; 1.0201x over previous
import functools

import jax
import jax.numpy as jnp
from jax import lax
from jax.experimental import pallas as pl
from jax.experimental.pallas import tpu as pltpu
from jax.experimental.pallas import tpu_sc as plsc

_NC = 2
_NS = 16


def _make_segsum(N, NP, E, w, K, NBUF):
    NW = _NC * _NS
    EPW = E // NW
    C = EPW // K
    assert EPW * NW == E and C * K == EPW and K % 8 == 0 and C >= NBUF
    RPT = NP // _NS
    assert RPT * _NS == NP and RPT % 8 == 0
    mesh = plsc.VectorSubcoreMesh(core_axis_name="c", subcore_axis_name="s")

    @functools.partial(
        pl.kernel,
        out_type=jax.ShapeDtypeStruct((NP, 128), jnp.float32),
        mesh=mesh,
        compiler_params=pltpu.CompilerParams(use_tc_tiling_on_sc=False),
        scratch_types=[
            pltpu.VMEM((C, K), jnp.int32),
            pltpu.VMEM((C, K), jnp.int32),
            pltpu.VMEM((NBUF, K, w), jnp.float32),
            pltpu.VMEM_SHARED((NP, w), jnp.float32),
        ] + [pltpu.SemaphoreType.DMA] * (2 * NBUF),
    )
    def segsum(table, src_idx, dst_idx, zeros, out, src_v, dst_v, rows_v,
               acc_sh, *sems):
        gsem = sems[:NBUF]
        ssem = sems[NBUF:]
        c = lax.axis_index("c")
        s = lax.axis_index("s")
        wid = c * _NS + s
        r0 = s * RPT
        zc = pltpu.async_copy(zeros.at[pl.ds(0, RPT), pl.ds(0, w)],
                              acc_sh.at[pl.ds(r0, RPT)], ssem[0])
        sc0 = pltpu.async_copy(src_idx.at[wid], src_v, gsem[0])
        dc0 = pltpu.async_copy(dst_idx.at[wid], dst_v, gsem[1])
        sc0.wait()
        dc0.wait()

        def gstart(b, i):
            pltpu.async_copy(table.at[src_v.at[i]], rows_v.at[b], gsem[b])

        def swait(b, i):
            pltpu.make_async_copy(rows_v.at[b], acc_sh.at[dst_v.at[i]],
                                  ssem[b]).wait()

        def step(b, i):
            pltpu.make_async_copy(table.at[src_v.at[i]], rows_v.at[b],
                                  gsem[b]).wait()
            pltpu.async_copy(rows_v.at[b], acc_sh.at[dst_v.at[i]],
                             ssem[b], add=True)
            nb = (b + 2) % NBUF

            @pl.when(i + 2 < C)
            def _():
                @pl.when(i >= NBUF - 2)
                def _():
                    swait(nb, i)

                gstart(nb, i + 2)

        gstart(0, 0)
        gstart(1, 1)
        zc.wait()
        plsc.subcore_barrier()

        @pl.loop(0, C // NBUF)
        def _grp(j):
            i0 = NBUF * j
            for t in range(NBUF):
                step(t, i0 + t)

        for t in range(C % NBUF):
            step(t, C - (C % NBUF) + t)

        for b in range(NBUF):
            swait(b, 0)

        plsc.subcore_barrier()
        pltpu.sync_copy(acc_sh.at[pl.ds(r0, RPT)],
                        out.at[pl.ds(r0, RPT), pl.ds(c * w, w)])

    return segsum


_N, _NP, _E = 10000, 10240, 320000
_R = 5000


def _make_l1pre(N, D, dout2, R):
    def body(x, wcat, out):
        out[...] = jnp.dot(x[...], wcat[...],
                           preferred_element_type=jnp.float32)

    return pl.pallas_call(
        body,
        grid=(N // R,),
        in_specs=[
            pl.BlockSpec((R, D), lambda i: (i, 0)),
            pl.BlockSpec((D, dout2), lambda i: (0, 0)),
        ],
        out_specs=pl.BlockSpec((R, dout2), lambda i: (i, 0)),
        out_shape=jax.ShapeDtypeStruct((N, dout2), jnp.float32),
    )


def _make_l1post(N, NP, dout, R):
    def body(a, pr, b, out):
        out[...] = (a[:, :dout] + a[:, dout:2 * dout]
                    + pr[:, dout:2 * dout] + b[...])

    return pl.pallas_call(
        body,
        grid=(N // R,),
        in_specs=[
            pl.BlockSpec((R, 128), lambda i: (i, 0)),
            pl.BlockSpec((R, 2 * dout), lambda i: (i, 0)),
            pl.BlockSpec((1, dout), lambda i: (0, 0)),
        ],
        out_specs=pl.BlockSpec((R, dout), lambda i: (i, 0)),
        out_shape=jax.ShapeDtypeStruct((N, dout), jnp.float32),
    )


def _make_post(N, NP, din, dout, R):
    def body(a, h, wrel2, wroot, b, out):
        out[...] = (jnp.dot(a[:, :2 * din], wrel2[...],
                            preferred_element_type=jnp.float32)
                    + jnp.dot(h[...], wroot[...],
                              preferred_element_type=jnp.float32)
                    + b[...])

    return pl.pallas_call(
        body,
        grid=(N // R,),
        in_specs=[
            pl.BlockSpec((R, 128), lambda i: (i, 0)),
            pl.BlockSpec((R, din), lambda i: (i, 0)),
            pl.BlockSpec((2 * din, dout), lambda i: (0, 0)),
            pl.BlockSpec((din, dout), lambda i: (0, 0)),
            pl.BlockSpec((1, dout), lambda i: (0, 0)),
        ],
        out_specs=pl.BlockSpec((R, dout), lambda i: (i, 0)),
        out_shape=jax.ShapeDtypeStruct((N, dout), jnp.float32),
    )


def _make_post5_pool(N, din, G, C, R):
    nblk = N // R
    D = 2 * din

    def body(a, h, wrel2, wroot, b, batch, wlin, blin, out, acc):
        i = pl.program_id(0)

        @pl.when(i == 0)
        def _():
            acc[...] = jnp.zeros_like(acc)

        h5 = (jnp.dot(a[...], wrel2[...], preferred_element_type=jnp.float32)
              + jnp.dot(h[...], wroot[...],
                        preferred_element_type=jnp.float32)
              + b[...])
        onehot = (lax.broadcasted_iota(jnp.int32, (G, R), 0)
                  == batch[0]).astype(jnp.float32)
        acc[...] += jnp.dot(onehot, h5, preferred_element_type=jnp.float32)

        @pl.when(i == nblk - 1)
        def _():
            out[...] = (jnp.dot(acc[...], wlin[...],
                                preferred_element_type=jnp.float32)
                        + blin[...])

    return pl.pallas_call(
        body,
        grid=(nblk,),
        in_specs=[
            pl.BlockSpec((R, 128), lambda i: (i, 0)),
            pl.BlockSpec((R, din), lambda i: (i, 0)),
            pl.BlockSpec((D, D), lambda i: (0, 0)),
            pl.BlockSpec((din, D), lambda i: (0, 0)),
            pl.BlockSpec((1, D), lambda i: (0, 0)),
            pl.BlockSpec((1, 1, R), lambda i: (i, 0, 0)),
            pl.BlockSpec((D, C), lambda i: (0, 0)),
            pl.BlockSpec((1, C), lambda i: (0, 0)),
        ],
        out_specs=pl.BlockSpec((G, C), lambda i: (0, 0)),
        out_shape=jax.ShapeDtypeStruct((G, C), jnp.float32),
        scratch_shapes=[pltpu.VMEM((G, D), jnp.float32)],
    )
_K_w = {8: (2000, 2), 16: (2000, 2), 32: (1000, 2), 64: (200, 3)}
_segsum_w = {w: _make_segsum(_N, _NP, _E, w, k, nb)
             for w, (k, nb) in _K_w.items()}
_l1pre = _make_l1pre(_N, 128, 16, _R)
_l1post = _make_l1post(_N, _NP, 8, _R)
_post_w = {
    8: _make_post(_N, _NP, 8, 16, _R),
    16: _make_post(_N, _NP, 16, 32, _R),
    32: _make_post(_N, _NP, 32, 64, _R),
}
_post5_pool = _make_post5_pool(_N, 64, 256, 10, _R)


def kernel(x, edge_index, batch, W_rel1, b_rel1, W_root1, W_rel2, b_rel2,
           W_root2, W_rel3, b_rel3, W_root3, W_rel4, b_rel4, W_root4,
           W_rel5, b_rel5, W_root5, W_lin, b_lin):
    idx3 = {}
    for w, (k, _nb) in _K_w.items():
        c = _E // 32 // k
        idx3[w] = (edge_index[0].reshape(32, c, k),
                   edge_index[1].reshape(32, c, k))
    zeros = jnp.zeros((_NP // _NS, 128), jnp.float32)

    pr = _l1pre(x, jnp.concatenate([W_rel1, W_root1], axis=1))
    A = _segsum_w[8](pr[:, :8], idx3[8][0], idx3[8][1], zeros)
    h = _l1post(A, pr, b_rel1.reshape(1, -1))

    for (W_rel, b_rel, W_root) in (
        (W_rel2, b_rel2, W_root2),
        (W_rel3, b_rel3, W_root3),
        (W_rel4, b_rel4, W_root4),
    ):
        w = h.shape[1]
        A = _segsum_w[w](h, idx3[w][0], idx3[w][1], zeros)
        wrel2 = jnp.concatenate([W_rel, W_rel], axis=0)
        h = _post_w[w](A, h, wrel2, W_root, b_rel.reshape(1, -1))

    A = _segsum_w[64](h, idx3[64][0], idx3[64][1], zeros)
    wrel2 = jnp.concatenate([W_rel5, W_rel5], axis=0)
    return _post5_pool(A, h, wrel2, W_root5, b_rel5.reshape(1, -1),
                       batch.reshape(_N // _R, 1, _R), W_lin,
                       b_lin.reshape(1, -1))
